# baseline (speedup 1.0000x reference)
import math

import jax
import jax.numpy as jnp
from jax import lax
from jax.experimental import pallas as pl
from jax.experimental.pallas import tpu as pltpu
from jax.experimental.pallas import tpu_sc as plsc

_N = 10000
_D = 256
_E = 160000
_EPS = 1e-5

_NPAD = 10240
_EPAD = 163840
_BLK = 256
_NBLK = _NPAD // _BLK
_HALF = 128

_CH = 128
_TILES = 16
_CHUNKS_DEG = _EPAD // 32 // _CH
_RPT = _NPAD // _TILES

_SCH = 128
_SW = 4
_SNWIN = 21
_SC = _SW * _SNWIN
_SEPT = _SC * _SCH
_EPAD_S = _SEPT * _TILES

_BNSCALE = 1.0 / math.sqrt(1.0 + _EPS)


def _deg_body(dst_hbm, c0_hbm, c1_hbm, hist, zbuf, ones, dsti):
    c = lax.axis_index("c")
    s = lax.axis_index("s")
    wid = c * _TILES + s

    def _z(i, carry):
        zbuf[i] = jnp.zeros((16,), jnp.float32)
        return carry

    lax.fori_loop(0, _RPT, _z, 0)

    def _o(i, carry):
        ones[i] = jnp.full((16,), 1.0, jnp.float32)
        return carry

    lax.fori_loop(0, _CH, _o, 0)

    pltpu.sync_copy(dst_hbm.at[wid], dsti)
    rs = pl.ds(s * _RPT, _RPT)
    pltpu.sync_copy(zbuf, hist.at[rs])
    plsc.subcore_barrier()

    for j in range(_CHUNKS_DEG):
        pltpu.sync_copy(ones, hist.at[dsti.at[j]], add=True)
    plsc.subcore_barrier()

    @pl.when(c == 0)
    def _():
        pltpu.sync_copy(hist.at[rs], c0_hbm.at[rs])

    @pl.when(c == 1)
    def _():
        pltpu.sync_copy(hist.at[rs], c1_hbm.at[rs])


def _spmm_body(hps, src_hbm, dst_hbm, r_st, acc, srcv0, srcv1,
               dstv0, dstv1, rows0, rows1, isem, g0, g1, s0, s1):
    c = lax.axis_index("c")
    s = lax.axis_index("s")
    rows = (rows0, rows1)
    gsem = (g0, g1)
    ssem = (s0, s1)
    slots = ((srcv0, dstv0), (srcv1, dstv1))

    rs = pl.ds(s * _RPT, _RPT)
    rs_g = pl.ds((c * _TILES + s) * _RPT, _RPT)

    i1 = pltpu.async_copy(src_hbm.at[c, s, 0], srcv0, isem)
    i2 = pltpu.async_copy(dst_hbm.at[s, 0], dstv0, isem)
    pltpu.sync_copy(hps.at[rs_g], acc.at[rs])
    i1.wait()
    i2.wait()
    gh = {0: pltpu.async_copy(hps.at[srcv0.at[0]], rows[0], gsem[0]),
          1: pltpu.async_copy(hps.at[srcv0.at[1]], rows[1], gsem[1])}
    plsc.subcore_barrier()

    stg = {}
    for w in range(_SNWIN):
        swin, dwin = slots[w % 2]
        nsrc, ndst = slots[1 - w % 2]
        for h in range(_SW // 2):
            j0 = _SW * w + 2 * h
            gh.pop(j0).wait()
            gh.pop(j0 + 1).wait()
            sa = pltpu.async_copy(rows[0], acc.at[dwin.at[2 * h]], ssem[0],
                                  add=True)
            sb = pltpu.async_copy(rows[1], acc.at[dwin.at[2 * h + 1]],
                                  ssem[1], add=True)
            if h == 0 and w + 1 < _SNWIN:
                stg[w + 1] = (
                    pltpu.async_copy(src_hbm.at[c, s, w + 1], nsrc, isem),
                    pltpu.async_copy(dst_hbm.at[s, w + 1], ndst, isem))
            sa.wait()
            sb.wait()
            if h == 0:
                gh[j0 + 2] = pltpu.async_copy(hps.at[swin.at[2]], rows[0],
                                              gsem[0])
                gh[j0 + 3] = pltpu.async_copy(hps.at[swin.at[3]], rows[1],
                                              gsem[1])
            elif w + 1 < _SNWIN:
                h1, h2 = stg.pop(w + 1)
                h1.wait()
                h2.wait()
                gh[j0 + 2] = pltpu.async_copy(hps.at[nsrc.at[0]], rows[0],
                                              gsem[0])
                gh[j0 + 3] = pltpu.async_copy(hps.at[nsrc.at[1]], rows[1],
                                              gsem[1])
    plsc.subcore_barrier()
    pltpu.sync_copy(acc.at[rs], r_st.at[rs_g])


def _sc_mesh():
    return plsc.VectorSubcoreMesh(core_axis_name="c", subcore_axis_name="s")


def _deg_call(dst_dg):
    f = pl.kernel(
        _deg_body,
        out_type=[jax.ShapeDtypeStruct((_NPAD, 16), jnp.float32)] * 2,
        mesh=_sc_mesh(),
        scratch_types=[
            pltpu.VMEM_SHARED((_NPAD, 16), jnp.float32),
            pltpu.VMEM((_RPT, 16), jnp.float32),
            pltpu.VMEM((_CH, 16), jnp.float32),
            pltpu.VMEM((_CHUNKS_DEG, _CH), jnp.int32),
        ],
    )
    return f(dst_dg)


def _spmm_call(hps_flat, src_both, dst_sp):
    f = pl.kernel(
        _spmm_body,
        out_type=jax.ShapeDtypeStruct((2 * _NPAD, _HALF), jnp.float32),
        mesh=_sc_mesh(),
        scratch_types=(
            [pltpu.VMEM_SHARED((_NPAD, _HALF), jnp.float32)]
            + [pltpu.VMEM((_SW, _SCH), jnp.int32)] * 4
            + [pltpu.VMEM((_SCH, _HALF), jnp.float32)] * 2
            + [pltpu.SemaphoreType.DMA] * 5
        ),
    )
    return f(hps_flat, src_both, dst_sp)


def _dinv(i, c0_ref, c1_ref):
    deg = c0_ref[:, 0:1] + c1_ref[:, 0:1] + 1.0
    row = i * _BLK + lax.broadcasted_iota(jnp.int32, (_BLK, 1), 0)
    return jnp.where(row < _N, lax.rsqrt(deg), 0.0)


def _tc1(x_ref, w_ref, c0_ref, c1_ref, hp_ref):
    dinv = _dinv(pl.program_id(0), c0_ref, c1_ref)
    h = jnp.dot(x_ref[...], w_ref[...], preferred_element_type=jnp.float32)
    hp = h * dinv
    hp_ref[0] = hp[:, :_HALF]
    hp_ref[1] = hp[:, _HALF:]


def _tc2(r_ref, c0_ref, c1_ref, b_ref, g_ref, be_ref, w_ref, hp_ref):
    dinv = _dinv(pl.program_id(0), c0_ref, c1_ref)
    r = jnp.concatenate([r_ref[0], r_ref[1]], axis=1)
    xb = (r * dinv + b_ref[...]) * _BNSCALE * g_ref[...] + be_ref[...]
    xb = jnp.maximum(xb, 0.0)
    h = jnp.dot(xb, w_ref[...], preferred_element_type=jnp.float32)
    hp = h * dinv
    hp_ref[0] = hp[:, :_HALF]
    hp_ref[1] = hp[:, _HALF:]


def _tc3(r_ref, c0_ref, c1_ref, b_ref, out_ref):
    dinv = _dinv(pl.program_id(0), c0_ref, c1_ref)
    r = jnp.concatenate([r_ref[0], r_ref[1]], axis=1)
    out_ref[...] = r * dinv + b_ref[...]


_ROWSPEC = pl.BlockSpec((_BLK, _D), lambda i: (i, 0))
_WSPEC = pl.BlockSpec((_D, _D), lambda i: (0, 0))
_CSPEC = pl.BlockSpec((_BLK, 16), lambda i: (i, 0))
_VSPEC = pl.BlockSpec((1, _D), lambda i: (0, 0))
_STSPEC = pl.BlockSpec((2, _BLK, _HALF), lambda i: (0, i, 0))
_STSHAPE = jax.ShapeDtypeStruct((2, _NPAD, _HALF), jnp.float32)


def _tc1_call(x, W, c0, c1):
    return pl.pallas_call(
        _tc1,
        grid=(_NBLK,),
        in_specs=[_ROWSPEC, _WSPEC, _CSPEC, _CSPEC],
        out_specs=_STSPEC,
        out_shape=_STSHAPE,
    )(x, W, c0, c1)


def _tc2_call(r_st, c0, c1, b, g, be, W):
    return pl.pallas_call(
        _tc2,
        grid=(_NBLK,),
        in_specs=[_STSPEC, _CSPEC, _CSPEC, _VSPEC, _VSPEC, _VSPEC, _WSPEC],
        out_specs=_STSPEC,
        out_shape=_STSHAPE,
    )(r_st, c0, c1, b, g, be, W)


def _tc3_call(r_st, c0, c1, b):
    return pl.pallas_call(
        _tc3,
        grid=(_NBLK,),
        in_specs=[_STSPEC, _CSPEC, _CSPEC, _VSPEC],
        out_specs=_ROWSPEC,
        out_shape=jax.ShapeDtypeStruct((_NPAD, _D), jnp.float32),
    )(r_st, c0, c1, b)


def kernel(x, edge_index, W1, b1, g1, be1, W2, b2, g2, be2, W3, b3):
    ei = edge_index.astype(jnp.int32)
    pad_s = _N + jnp.arange(_EPAD_S - _E, dtype=jnp.int32) % (_NPAD - _N)
    src_sp = jnp.concatenate([ei[0], pad_s]).reshape(_TILES, _SNWIN, _SW,
                                                     _SCH)
    dst_sp = jnp.concatenate([ei[1], pad_s]).reshape(_TILES, _SNWIN, _SW,
                                                     _SCH)
    pad_d = jnp.full((_EPAD - _E,), _NPAD - 1, jnp.int32)
    dst_dg = jnp.concatenate([ei[1], pad_d]).reshape(32, _CHUNKS_DEG, _CH)
    x_pad = jnp.pad(x, ((0, _NPAD - _N), (0, 0)))
    b1r, g1r, be1r = b1.reshape(1, _D), g1.reshape(1, _D), be1.reshape(1, _D)
    b2r, g2r, be2r = b2.reshape(1, _D), g2.reshape(1, _D), be2.reshape(1, _D)
    b3r = b3.reshape(1, _D)

    src_both = jnp.stack([src_sp, src_sp + _NPAD])

    c0, c1 = _deg_call(dst_dg)
    hp = _tc1_call(x_pad, W1, c0, c1)
    r = _spmm_call(hp.reshape(2 * _NPAD, _HALF), src_both, dst_sp)
    hp = _tc2_call(r.reshape(2, _NPAD, _HALF), c0, c1, b1r, g1r, be1r, W2)
    r = _spmm_call(hp.reshape(2 * _NPAD, _HALF), src_both, dst_sp)
    hp = _tc2_call(r.reshape(2, _NPAD, _HALF), c0, c1, b2r, g2r, be2r, W3)
    r = _spmm_call(hp.reshape(2 * _NPAD, _HALF), src_both, dst_sp)
    out = _tc3_call(r.reshape(2, _NPAD, _HALF), c0, c1, b3r)
    return out[:_N]
